# baseline (speedup 1.0000x reference)
import functools

import numpy as np
import jax
import jax.numpy as jnp
from jax import lax
from jax.experimental import pallas as pl
from jax.experimental.pallas import tpu as pltpu
from jax.experimental.pallas import tpu_sc as plsc

N = 10000
E = 160000
NC, NS = 2, 16
NW = NC * NS
K = 125
C = E // (NW * K)
PW = C * K
CPB = 8
NB = C // CPB
NPAD = 10240
RP = NPAD // NS
DOUT = 16
EP = E // 8
EPS = 1e-5
_MESH = dict(core_axis_name="c", subcore_axis_name="s")


def _gather16(tables, idx_w):
    nt = len(tables)
    mesh = plsc.VectorSubcoreMesh(**_MESH)

    @functools.partial(
        pl.kernel,
        out_type=[jax.ShapeDtypeStruct((E, DOUT), jnp.float32)] * nt,
        mesh=mesh,
        compiler_params=pltpu.CompilerParams(use_tc_tiling_on_sc=False),
        scratch_types=[pltpu.VMEM((C, K), jnp.int32)]
        + [pltpu.VMEM((CPB * K, DOUT), jnp.float32)] * nt
        + [pltpu.SemaphoreType.DMA],
    )
    def gk(*refs):
        tabs = refs[:nt]
        idx_hbm = refs[nt]
        outs = refs[nt + 1:2 * nt + 1]
        idx_v = refs[2 * nt + 1]
        bufs = refs[2 * nt + 2:3 * nt + 2]
        sem = refs[3 * nt + 2]
        wid = lax.axis_index("s") * NC + lax.axis_index("c")
        base = wid * PW
        pltpu.sync_copy(idx_hbm.at[wid], idx_v)

        @pl.loop(0, NB)
        def _(cc):
            cps = []
            for t in range(CPB):
                for tab, buf in zip(tabs, bufs):
                    cps.append(pltpu.async_copy(
                        tab.at[idx_v.at[cc * CPB + t]],
                        buf.at[pl.ds(t * K, K)], sem))
            for cp in cps:
                cp.wait()
            for buf, out in zip(bufs, outs):
                pltpu.sync_copy(
                    buf, out.at[pl.ds(base + cc * (CPB * K), CPB * K)])

    res = gk(*tables, idx_w)
    if not isinstance(res, (list, tuple)):
        res = [res]
    return [r.reshape(EP, 128) for r in res]


def _scatter(msg_p, idx_w, zinit):
    mesh = plsc.VectorSubcoreMesh(**_MESH)

    @functools.partial(
        pl.kernel,
        out_type=jax.ShapeDtypeStruct((NC, NPAD, DOUT), jnp.float32),
        mesh=mesh,
        compiler_params=pltpu.CompilerParams(use_tc_tiling_on_sc=False),
        scratch_types=[
            pltpu.VMEM((C, K), jnp.int32),
            pltpu.VMEM((PW, DOUT), jnp.float32),
            pltpu.VMEM_SHARED((NPAD, DOUT), jnp.float32),
            pltpu.SemaphoreType.DMA,
        ],
    )
    def sk(msg_hbm, idx_hbm, zero_hbm, out_hbm, idx_v, msg_v, acc_sh, sem):
        cid = lax.axis_index("c")
        sid = lax.axis_index("s")
        wid = sid * NC + cid
        row0 = sid * RP
        pltpu.sync_copy(zero_hbm.at[pl.ds(row0, RP)], acc_sh.at[pl.ds(row0, RP)])
        plsc.subcore_barrier()
        pltpu.sync_copy(msg_hbm.at[pl.ds(wid * PW, PW)], msg_v)
        pltpu.sync_copy(idx_hbm.at[wid], idx_v)

        @pl.loop(0, NB)
        def _(cc):
            cps = []
            for t in range(CPB):
                j = cc * CPB + t
                cps.append(pltpu.async_copy(
                    msg_v.at[pl.ds(j * K, K)], acc_sh.at[idx_v.at[j]], sem,
                    add=True))
            for cp in cps:
                cp.wait()

        plsc.subcore_barrier()
        pltpu.sync_copy(acc_sh.at[pl.ds(row0, RP)], out_hbm.at[cid, pl.ds(row0, RP)])

    return sk(msg_p.reshape(E, DOUT), idx_w, zinit)


def _eq_consts():
    eqs_np = np.zeros((8 * 128, DOUT), np.float32)
    for q in range(8):
        for c in range(DOUT):
            eqs_np[q * 128 + q * DOUT + c, c] = 1.0
    eqt_np = np.concatenate(
        [eqs_np[q * 128:(q + 1) * 128].T for q in range(8)], axis=1)
    return jnp.asarray(eqs_np), jnp.asarray(eqt_np)


def _msg(ea_p, xps, Wa, ba, Wb, bb, Rs, Sm, extra, block_e=16000):
    G = E // block_e
    PR = block_e // 8
    Ha = Wa.shape[1]
    nx = len(xps)
    eqs, eqt = _eq_consts()
    f32 = jnp.float32
    waq = jnp.dot(eqs, Wa, preferred_element_type=f32)
    rq = [jnp.dot(eqs, r, preferred_element_type=f32) for r in Rs]
    sq = jnp.dot(Sm, eqt, preferred_element_type=f32)
    exp = jnp.tile(extra, (1, 8))

    def body(*refs):
        ea_ref = refs[0]
        xp_refs = refs[1:1 + nx]
        (waq_ref, ba_ref, wb_ref, bb_ref) = refs[1 + nx:5 + nx]
        rq_refs = refs[5 + nx:5 + 2 * nx]
        (sq_ref, ex_ref, out_ref) = refs[5 + 2 * nx:]
        bf16 = jnp.bfloat16

        def dot(a, b):
            return jnp.dot(a.astype(bf16), b.astype(bf16),
                           preferred_element_type=f32)

        eap = ea_ref[...]
        xpv = [r[...] for r in xp_refs]
        acc = ex_ref[...] + jnp.zeros((PR, 128), f32)
        for q in range(8):
            h = jnp.maximum(
                dot(eap, waq_ref[pl.ds(q * 128, 128), :]) + ba_ref[...], 0.0)
            we = dot(h, wb_ref[...]) + bb_ref[...]
            xt = dot(xpv[0], rq_refs[0][pl.ds(q * 128, 128), :])
            for t in range(1, nx):
                xt = xt + dot(xpv[t], rq_refs[t][pl.ds(q * 128, 128), :])
            acc = acc + dot(we * xt, sq_ref[:, pl.ds(q * 128, 128)])
        out_ref[...] = acc

    full = lambda shape: pl.BlockSpec(shape, lambda i: (0, 0))
    return pl.pallas_call(
        body,
        grid=(G,),
        in_specs=[pl.BlockSpec((PR, 128), lambda i: (i, 0))] * (1 + nx)
        + [full((8 * 128, Ha)), full((1, Ha)), full((Ha, Ha)), full((1, Ha))]
        + [full((8 * 128, Ha))] * nx
        + [full((Ha, 8 * 128)), full((1, 128))],
        out_specs=pl.BlockSpec((PR, 128), lambda i: (i, 0)),
        out_shape=jax.ShapeDtypeStruct((EP, 128), jnp.float32),
    )(ea_p, *xps, waq, ba, Wb, bb, *rq, sq, exp)


def _post(parts, inv_in, x_cur, root, bias, g, be, c_in, c_out, with_cnt):
    outs = [jax.ShapeDtypeStruct((N, DOUT), jnp.float32)]
    if with_cnt:
        outs.append(jax.ShapeDtypeStruct((N, 1), jnp.float32))

    def body(*refs):
        if with_cnt:
            parts_ref, x_ref, root_ref, bias_ref, g_ref, be_ref, out_ref, inv_ref = refs
        else:
            parts_ref, invin_ref, x_ref, root_ref, bias_ref, g_ref, be_ref, out_ref = refs
        acc = parts_ref[0:N, :] + parts_ref[NPAD:NPAD + N, :]
        if with_cnt:
            inv = 1.0 / jnp.maximum(acc[:, c_out:c_out + 1], 1.0)
            inv_ref[...] = inv
        else:
            inv = invin_ref[...]
        h = (acc[:, 0:c_out] * inv
             + jnp.dot(x_ref[...][:, 0:c_in], root_ref[...],
                       preferred_element_type=jnp.float32)
             + bias_ref[...])
        mu = jnp.mean(h, axis=0, keepdims=True)
        var = jnp.mean((h - mu) ** 2, axis=0, keepdims=True)
        y = g_ref[...] * (h - mu) * lax.rsqrt(var + EPS) + be_ref[...]
        y = jnp.maximum(y, 0.0)
        if c_out < DOUT:
            y = jnp.concatenate(
                [y, jnp.zeros((N, DOUT - c_out), jnp.float32)], axis=1)
        out_ref[...] = y

    ins = [parts] + ([] if with_cnt else [inv_in]) + [x_cur, root, bias, g, be]
    res = pl.pallas_call(body, out_shape=outs)(*ins)
    return res if with_cnt else res[0]


def _mk_RS(c_in, c_out):
    ha = c_in * c_out
    fp = 32 if c_in == 32 else DOUT
    rm = np.zeros((fp, ha), np.float32)
    sm = np.zeros((ha, DOUT), np.float32)
    for i in range(c_in):
        for o in range(c_out):
            rm[i, i * c_out + o] = 1.0
            sm[i * c_out + o, o] = 1.0
    return jnp.asarray(rm), jnp.asarray(sm)


def kernel(x, edge_index, edge_attr, W1a, b1a, W1b, b1b, root1, bias1, g1, be1,
           W2a, b2a, W2b, b2b, root2, bias2, g2, be2,
           W3a, b3a, W3b, b3b, root3, bias3, g3, be3):
    src = edge_index[0].astype(jnp.int32).reshape(NW, C, K)
    dst = edge_index[1].astype(jnp.int32).reshape(NW, C, K)
    zinit = jnp.zeros((NPAD, DOUT), jnp.float32)
    ea_p = edge_attr.reshape(EP, 128)

    r1, s1 = _mk_RS(32, 8)
    r2, s2 = _mk_RS(8, 4)
    r3, s3 = _mk_RS(4, 16)
    ex1 = np.zeros((1, DOUT), np.float32)
    ex1[0, 8] = 1.0
    ex1 = jnp.asarray(ex1)
    ex0 = jnp.zeros((1, DOUT), jnp.float32)

    def row(v):
        return v.reshape(1, -1)

    xa, xb = _gather16([x[:, :16], x[:, 16:]], src)
    msg = _msg(ea_p, [xa, xb], W1a, row(b1a), W1b, row(b1b),
               [r1[:16], r1[16:]], s1, ex1)
    parts = _scatter(msg, dst, zinit)
    h1, invc = _post(parts.reshape(2 * NPAD, DOUT), None, x, root1, row(bias1),
                     row(g1), row(be1), 32, 8, True)

    xj, = _gather16([h1], src)
    msg = _msg(ea_p, [xj], W2a, row(b2a), W2b, row(b2b), [r2], s2, ex0)
    parts = _scatter(msg, dst, zinit)
    h2 = _post(parts.reshape(2 * NPAD, DOUT), invc, h1, root2, row(bias2),
               row(g2), row(be2), 8, 4, False)

    xj, = _gather16([h2], src)
    msg = _msg(ea_p, [xj], W3a, row(b3a), W3b, row(b3b), [r3], s3, ex0)
    parts = _scatter(msg, dst, zinit)
    h3 = _post(parts.reshape(2 * NPAD, DOUT), invc, h2, root3, row(bias3),
               row(g3), row(be3), 4, 16, False)
    return h3


# speedup vs baseline: 1.2020x; 1.2020x over previous
import functools

import numpy as np
import jax
import jax.numpy as jnp
from jax import lax
from jax.experimental import pallas as pl
from jax.experimental.pallas import tpu as pltpu
from jax.experimental.pallas import tpu_sc as plsc

N = 10000
E = 160000
NC, NS = 2, 16
NW = NC * NS
K = 125
C = E // (NW * K)
PW = C * K
CPB = 8
NB = C // CPB
NPAD = 10240
RP = NPAD // NS
DOUT = 16
EP = E // 8
EPS = 1e-5
_MESH = dict(core_axis_name="c", subcore_axis_name="s")


def _gather16(tables, idx_w):
    nt = len(tables)
    mesh = plsc.VectorSubcoreMesh(**_MESH)

    @functools.partial(
        pl.kernel,
        out_type=[jax.ShapeDtypeStruct((E, DOUT), jnp.float32)] * nt,
        mesh=mesh,
        compiler_params=pltpu.CompilerParams(use_tc_tiling_on_sc=False),
        scratch_types=[pltpu.VMEM((C, K), jnp.int32)]
        + [pltpu.VMEM((CPB * K, DOUT), jnp.float32)] * nt
        + [pltpu.SemaphoreType.DMA],
    )
    def gk(*refs):
        tabs = refs[:nt]
        idx_hbm = refs[nt]
        outs = refs[nt + 1:2 * nt + 1]
        idx_v = refs[2 * nt + 1]
        bufs = refs[2 * nt + 2:3 * nt + 2]
        sem = refs[3 * nt + 2]
        wid = lax.axis_index("s") * NC + lax.axis_index("c")
        base = wid * PW
        pltpu.sync_copy(idx_hbm.at[wid], idx_v)

        @pl.loop(0, NB)
        def _(cc):
            cps = []
            for t in range(CPB):
                for tab, buf in zip(tabs, bufs):
                    cps.append(pltpu.async_copy(
                        tab.at[idx_v.at[cc * CPB + t]],
                        buf.at[pl.ds(t * K, K)], sem))
            for cp in cps:
                cp.wait()
            for buf, out in zip(bufs, outs):
                pltpu.sync_copy(
                    buf, out.at[pl.ds(base + cc * (CPB * K), CPB * K)])

    res = gk(*tables, idx_w)
    if not isinstance(res, (list, tuple)):
        res = [res]
    return [r.reshape(EP, 128) for r in res]


def _scatter(msg_p, idx_w, zinit):
    mesh = plsc.VectorSubcoreMesh(**_MESH)

    @functools.partial(
        pl.kernel,
        out_type=jax.ShapeDtypeStruct((NC, NPAD, DOUT), jnp.float32),
        mesh=mesh,
        compiler_params=pltpu.CompilerParams(use_tc_tiling_on_sc=False),
        scratch_types=[
            pltpu.VMEM((C, K), jnp.int32),
            pltpu.VMEM((PW, DOUT), jnp.float32),
            pltpu.VMEM_SHARED((NPAD, DOUT), jnp.float32),
            pltpu.SemaphoreType.DMA,
        ],
    )
    def sk(msg_hbm, idx_hbm, zero_hbm, out_hbm, idx_v, msg_v, acc_sh, sem):
        cid = lax.axis_index("c")
        sid = lax.axis_index("s")
        wid = sid * NC + cid
        row0 = sid * RP
        pltpu.sync_copy(zero_hbm.at[pl.ds(row0, RP)], acc_sh.at[pl.ds(row0, RP)])
        plsc.subcore_barrier()
        pltpu.sync_copy(msg_hbm.at[pl.ds(wid * PW, PW)], msg_v)
        pltpu.sync_copy(idx_hbm.at[wid], idx_v)

        @pl.loop(0, NB)
        def _(cc):
            cps = []
            for t in range(CPB):
                j = cc * CPB + t
                cps.append(pltpu.async_copy(
                    msg_v.at[pl.ds(j * K, K)], acc_sh.at[idx_v.at[j]], sem,
                    add=True))
            for cp in cps:
                cp.wait()

        plsc.subcore_barrier()
        pltpu.sync_copy(acc_sh.at[pl.ds(row0, RP)], out_hbm.at[cid, pl.ds(row0, RP)])

    return sk(msg_p.reshape(E, DOUT), idx_w, zinit)


def _eq_consts():
    eqs_np = np.zeros((8 * 128, DOUT), np.float32)
    for q in range(8):
        for c in range(DOUT):
            eqs_np[q * 128 + q * DOUT + c, c] = 1.0
    eqt_np = np.concatenate(
        [eqs_np[q * 128:(q + 1) * 128].T for q in range(8)], axis=1)
    return jnp.asarray(eqs_np), jnp.asarray(eqt_np)


def _msg(ea_p, xps, Wa, ba, Wb, bb, Rs, Sm, extra, block_e=16000):
    G = E // block_e
    PR = block_e // 8
    Ha = Wa.shape[1]
    nx = len(xps)
    eqs, eqt = _eq_consts()
    f32 = jnp.float32
    dj = functools.partial(jnp.dot, preferred_element_type=f32)
    exp = jnp.tile(extra, (1, 8))

    if Ha <= 64:
        HW = 8 * Ha
        waw = jnp.concatenate(
            [dj(eqs[q * 128:(q + 1) * 128], Wa) for q in range(8)], axis=1)
        rqw = [jnp.concatenate(
            [dj(eqs[q * 128:(q + 1) * 128], r) for q in range(8)], axis=1)
            for r in Rs]
        wbd = jnp.kron(jnp.eye(8, dtype=f32), Wb)
        sqw = jnp.concatenate(
            [dj(Sm, eqt[:, q * 128:(q + 1) * 128]) for q in range(8)], axis=0)
        baw = jnp.tile(ba, (1, 8))
        dsq = jnp.dot(jnp.diag(bb[0]), Sm, preferred_element_type=f32)
        dsqw = jnp.concatenate(
            [dj(dsq, eqt[:, q * 128:(q + 1) * 128]) for q in range(8)], axis=0)

        def body(*refs):
            ea_ref = refs[0]
            xp_refs = refs[1:1 + nx]
            (waw_ref, baw_ref, wbd_ref) = refs[1 + nx:4 + nx]
            rq_refs = refs[4 + nx:4 + 2 * nx]
            (sqw_ref, dsq_ref, ex_ref, out_ref) = refs[4 + 2 * nx:]
            eap = ea_ref[...]
            h = jnp.maximum(dj(eap, waw_ref[...]) + baw_ref[...], 0.0)
            we = dj(h, wbd_ref[...])
            xt = dj(xp_refs[0][...], rq_refs[0][...])
            for t in range(1, nx):
                xt = xt + dj(xp_refs[t][...], rq_refs[t][...])
            out_ref[...] = (ex_ref[...] + dj(we * xt, sqw_ref[...])
                            + dj(xt, dsq_ref[...]))

        full = lambda shape: pl.BlockSpec(shape, lambda i: (0, 0))
        return pl.pallas_call(
            body,
            grid=(G,),
            in_specs=[pl.BlockSpec((PR, 128), lambda i: (i, 0))] * (1 + nx)
            + [full((128, HW)), full((1, HW)), full((HW, HW))]
            + [full((128, HW))] * nx
            + [full((HW, 128)), full((HW, 128)), full((1, 128))],
            out_specs=pl.BlockSpec((PR, 128), lambda i: (i, 0)),
            out_shape=jax.ShapeDtypeStruct((EP, 128), jnp.float32),
        )(ea_p, *xps, waw, baw, wbd, *rqw, sqw, dsqw, exp)

    waq = dj(eqs, Wa)
    rq = [dj(eqs, r) for r in Rs]
    sq = dj(Sm, eqt)
    dsq = dj(jnp.dot(jnp.diag(bb[0]), Sm,
                     preferred_element_type=f32), eqt)

    def body(*refs):
        ea_ref = refs[0]
        xp_refs = refs[1:1 + nx]
        (waq_ref, ba_ref, wb_ref) = refs[1 + nx:4 + nx]
        rq_refs = refs[4 + nx:4 + 2 * nx]
        (sq_ref, dsq_ref, ex_ref, out_ref) = refs[4 + 2 * nx:]
        dot = functools.partial(jnp.dot, preferred_element_type=f32)
        eap = ea_ref[...]
        xpv = [r[...] for r in xp_refs]
        acc = ex_ref[...] + jnp.zeros((PR, 128), f32)
        for q in range(8):
            h = jnp.maximum(
                dot(eap, waq_ref[pl.ds(q * 128, 128), :]) + ba_ref[...], 0.0)
            we = dot(h, wb_ref[...])
            xt = dot(xpv[0], rq_refs[0][pl.ds(q * 128, 128), :])
            for t in range(1, nx):
                xt = xt + dot(xpv[t], rq_refs[t][pl.ds(q * 128, 128), :])
            acc = (acc + dot(we * xt, sq_ref[:, pl.ds(q * 128, 128)])
                   + dot(xt, dsq_ref[:, pl.ds(q * 128, 128)]))
        out_ref[...] = acc

    full = lambda shape: pl.BlockSpec(shape, lambda i: (0, 0))
    return pl.pallas_call(
        body,
        grid=(G,),
        in_specs=[pl.BlockSpec((PR, 128), lambda i: (i, 0))] * (1 + nx)
        + [full((8 * 128, Ha)), full((1, Ha)), full((Ha, Ha))]
        + [full((8 * 128, Ha))] * nx
        + [full((Ha, 8 * 128)), full((Ha, 8 * 128)), full((1, 128))],
        out_specs=pl.BlockSpec((PR, 128), lambda i: (i, 0)),
        out_shape=jax.ShapeDtypeStruct((EP, 128), jnp.float32),
    )(ea_p, *xps, waq, ba, Wb, *rq, sq, dsq, exp)


def _post(parts, inv_in, x_cur, root, bias, g, be, c_in, c_out, with_cnt):
    outs = [jax.ShapeDtypeStruct((N, DOUT), jnp.float32)]
    if with_cnt:
        outs.append(jax.ShapeDtypeStruct((N, 1), jnp.float32))

    def body(*refs):
        if with_cnt:
            parts_ref, x_ref, root_ref, bias_ref, g_ref, be_ref, out_ref, inv_ref = refs
        else:
            parts_ref, invin_ref, x_ref, root_ref, bias_ref, g_ref, be_ref, out_ref = refs
        acc = parts_ref[0:N, :] + parts_ref[NPAD:NPAD + N, :]
        if with_cnt:
            inv = 1.0 / jnp.maximum(acc[:, c_out:c_out + 1], 1.0)
            inv_ref[...] = inv
        else:
            inv = invin_ref[...]
        h = (acc[:, 0:c_out] * inv
             + jnp.dot(x_ref[...][:, 0:c_in], root_ref[...],
                       preferred_element_type=jnp.float32)
             + bias_ref[...])
        mu = jnp.mean(h, axis=0, keepdims=True)
        var = jnp.mean((h - mu) ** 2, axis=0, keepdims=True)
        y = g_ref[...] * (h - mu) * lax.rsqrt(var + EPS) + be_ref[...]
        y = jnp.maximum(y, 0.0)
        if c_out < DOUT:
            y = jnp.concatenate(
                [y, jnp.zeros((N, DOUT - c_out), jnp.float32)], axis=1)
        out_ref[...] = y

    ins = [parts] + ([] if with_cnt else [inv_in]) + [x_cur, root, bias, g, be]
    res = pl.pallas_call(body, out_shape=outs)(*ins)
    return res if with_cnt else res[0]


def _mk_RS(c_in, c_out):
    ha = c_in * c_out
    fp = 32 if c_in == 32 else DOUT
    rm = np.zeros((fp, ha), np.float32)
    sm = np.zeros((ha, DOUT), np.float32)
    for i in range(c_in):
        for o in range(c_out):
            rm[i, i * c_out + o] = 1.0
            sm[i * c_out + o, o] = 1.0
    return jnp.asarray(rm), jnp.asarray(sm)


def kernel(x, edge_index, edge_attr, W1a, b1a, W1b, b1b, root1, bias1, g1, be1,
           W2a, b2a, W2b, b2b, root2, bias2, g2, be2,
           W3a, b3a, W3b, b3b, root3, bias3, g3, be3):
    src = edge_index[0].astype(jnp.int32).reshape(NW, C, K)
    dst = edge_index[1].astype(jnp.int32).reshape(NW, C, K)
    zinit = jnp.zeros((NPAD, DOUT), jnp.float32)
    ea_p = edge_attr.reshape(EP, 128)

    r1, s1 = _mk_RS(32, 8)
    r2, s2 = _mk_RS(8, 4)
    r3, s3 = _mk_RS(4, 16)
    ex1 = np.zeros((1, DOUT), np.float32)
    ex1[0, 8] = 1.0
    ex1 = jnp.asarray(ex1)
    ex0 = jnp.zeros((1, DOUT), jnp.float32)

    def row(v):
        return v.reshape(1, -1)

    xa, xb = _gather16([x[:, :16], x[:, 16:]], src)
    msg = _msg(ea_p, [xa, xb], W1a, row(b1a), W1b, row(b1b),
               [r1[:16], r1[16:]], s1, ex1)
    parts = _scatter(msg, dst, zinit)
    h1, invc = _post(parts.reshape(2 * NPAD, DOUT), None, x, root1, row(bias1),
                     row(g1), row(be1), 32, 8, True)

    xj, = _gather16([h1], src)
    msg = _msg(ea_p, [xj], W2a, row(b2a), W2b, row(b2b), [r2], s2, ex0)
    parts = _scatter(msg, dst, zinit)
    h2 = _post(parts.reshape(2 * NPAD, DOUT), invc, h1, root2, row(bias2),
               row(g2), row(be2), 8, 4, False)

    xj, = _gather16([h2], src)
    msg = _msg(ea_p, [xj], W3a, row(b3a), W3b, row(b3b), [r3], s3, ex0)
    parts = _scatter(msg, dst, zinit)
    h3 = _post(parts.reshape(2 * NPAD, DOUT), invc, h2, root3, row(bias3),
               row(g3), row(be3), 4, 16, False)
    return h3


# speedup vs baseline: 1.2981x; 1.0799x over previous
import functools

import numpy as np
import jax
import jax.numpy as jnp
from jax import lax
from jax.experimental import pallas as pl
from jax.experimental.pallas import tpu as pltpu
from jax.experimental.pallas import tpu_sc as plsc

N = 10000
E = 160000
NC, NS = 2, 16
NW = NC * NS
K = 125
C = E // (NW * K)
PW = C * K
CPB = 8
NB = C // CPB
NPAD = 10240
RP = NPAD // NS
DOUT = 16
EP = E // 8
EPS = 1e-5
_MESH = dict(core_axis_name="c", subcore_axis_name="s")


def _gather16(tables, idx_w):
    nt = len(tables)
    mesh = plsc.VectorSubcoreMesh(**_MESH)

    @functools.partial(
        pl.kernel,
        out_type=[jax.ShapeDtypeStruct((E, DOUT), jnp.float32)] * nt,
        mesh=mesh,
        compiler_params=pltpu.CompilerParams(use_tc_tiling_on_sc=False),
        scratch_types=[pltpu.VMEM((C, K), jnp.int32)]
        + [pltpu.VMEM((CPB * K, DOUT), jnp.float32)] * nt
        + [pltpu.SemaphoreType.DMA],
    )
    def gk(*refs):
        tabs = refs[:nt]
        idx_hbm = refs[nt]
        outs = refs[nt + 1:2 * nt + 1]
        idx_v = refs[2 * nt + 1]
        bufs = refs[2 * nt + 2:3 * nt + 2]
        sem = refs[3 * nt + 2]
        wid = lax.axis_index("s") * NC + lax.axis_index("c")
        base = wid * PW
        pltpu.sync_copy(idx_hbm.at[wid], idx_v)

        @pl.loop(0, NB)
        def _(cc):
            cps = []
            for t in range(CPB):
                for tab, buf in zip(tabs, bufs):
                    cps.append(pltpu.async_copy(
                        tab.at[idx_v.at[cc * CPB + t]],
                        buf.at[pl.ds(t * K, K)], sem))
            for cp in cps:
                cp.wait()
            for buf, out in zip(bufs, outs):
                pltpu.sync_copy(
                    buf, out.at[pl.ds(base + cc * (CPB * K), CPB * K)])

    res = gk(*tables, idx_w)
    if not isinstance(res, (list, tuple)):
        res = [res]
    return [r.reshape(EP, 128) for r in res]


def _scatter(msg_p, idx_w, zinit):
    mesh = plsc.VectorSubcoreMesh(**_MESH)

    @functools.partial(
        pl.kernel,
        out_type=jax.ShapeDtypeStruct((NC, NPAD, DOUT), jnp.float32),
        mesh=mesh,
        compiler_params=pltpu.CompilerParams(use_tc_tiling_on_sc=False),
        scratch_types=[
            pltpu.VMEM((C, K), jnp.int32),
            pltpu.VMEM((PW, DOUT), jnp.float32),
            pltpu.VMEM_SHARED((NPAD, DOUT), jnp.float32),
            pltpu.SemaphoreType.DMA,
        ],
    )
    def sk(msg_hbm, idx_hbm, zero_hbm, out_hbm, idx_v, msg_v, acc_sh, sem):
        cid = lax.axis_index("c")
        sid = lax.axis_index("s")
        wid = sid * NC + cid
        row0 = sid * RP
        pltpu.sync_copy(zero_hbm.at[pl.ds(row0, RP)], acc_sh.at[pl.ds(row0, RP)])
        plsc.subcore_barrier()
        pltpu.sync_copy(msg_hbm.at[pl.ds(wid * PW, PW)], msg_v)
        pltpu.sync_copy(idx_hbm.at[wid], idx_v)

        @pl.loop(0, NB)
        def _(cc):
            cps = []
            for t in range(CPB):
                j = cc * CPB + t
                cps.append(pltpu.async_copy(
                    msg_v.at[pl.ds(j * K, K)], acc_sh.at[idx_v.at[j]], sem,
                    add=True))
            for cp in cps:
                cp.wait()

        plsc.subcore_barrier()
        pltpu.sync_copy(acc_sh.at[pl.ds(row0, RP)], out_hbm.at[cid, pl.ds(row0, RP)])

    return sk(msg_p.reshape(E, DOUT), idx_w, zinit)


def _eq_consts():
    eqs_np = np.zeros((8 * 128, DOUT), np.float32)
    for q in range(8):
        for c in range(DOUT):
            eqs_np[q * 128 + q * DOUT + c, c] = 1.0
    eqt_np = np.concatenate(
        [eqs_np[q * 128:(q + 1) * 128].T for q in range(8)], axis=1)
    return jnp.asarray(eqs_np), jnp.asarray(eqt_np)


def _msg(ea_p, xps, Wa, ba, Wb, bb, Rs, Sm, extra, block_e=16000):
    G = E // block_e
    PR = block_e // 8
    Ha = Wa.shape[1]
    nx = len(xps)
    eqs, eqt = _eq_consts()
    f32 = jnp.float32
    dj = functools.partial(jnp.dot, preferred_element_type=f32)
    exp = jnp.tile(extra, (1, 8))

    if Ha <= 64:
        HW = 8 * Ha
        waw = jnp.concatenate(
            [dj(eqs[q * 128:(q + 1) * 128], Wa) for q in range(8)], axis=1)
        rqw = [jnp.concatenate(
            [dj(eqs[q * 128:(q + 1) * 128], r) for q in range(8)], axis=1)
            for r in Rs]
        wbd = jnp.kron(jnp.eye(8, dtype=f32), Wb)
        sqw = jnp.concatenate(
            [dj(Sm, eqt[:, q * 128:(q + 1) * 128]) for q in range(8)], axis=0)
        baw = jnp.tile(ba, (1, 8))
        dsq = jnp.dot(jnp.diag(bb[0]), Sm, preferred_element_type=f32)
        dsqw = jnp.concatenate(
            [dj(dsq, eqt[:, q * 128:(q + 1) * 128]) for q in range(8)], axis=0)

        def body(*refs):
            ea_ref = refs[0]
            xp_refs = refs[1:1 + nx]
            (waw_ref, baw_ref, wbd_ref) = refs[1 + nx:4 + nx]
            rq_refs = refs[4 + nx:4 + 2 * nx]
            (sqw_ref, dsq_ref, ex_ref, out_ref) = refs[4 + 2 * nx:]
            eap = ea_ref[...]
            h = jnp.maximum(dj(eap, waw_ref[...]) + baw_ref[...], 0.0)
            we = dj(h, wbd_ref[...])
            xt = dj(xp_refs[0][...], rq_refs[0][...])
            for t in range(1, nx):
                xt = xt + dj(xp_refs[t][...], rq_refs[t][...])
            out_ref[...] = (ex_ref[...] + dj(we * xt, sqw_ref[...])
                            + dj(xt, dsq_ref[...]))

        full = lambda shape: pl.BlockSpec(shape, lambda i: (0, 0))
        return pl.pallas_call(
            body,
            grid=(G,),
            in_specs=[pl.BlockSpec((PR, 128), lambda i: (i, 0))] * (1 + nx)
            + [full((128, HW)), full((1, HW)), full((HW, HW))]
            + [full((128, HW))] * nx
            + [full((HW, 128)), full((HW, 128)), full((1, 128))],
            out_specs=pl.BlockSpec((PR, 128), lambda i: (i, 0)),
            out_shape=jax.ShapeDtypeStruct((EP, 128), jnp.float32),
        )(ea_p, *xps, waw, baw, wbd, *rqw, sqw, dsqw, exp)

    waq = dj(eqs, Wa)
    rq = [dj(eqs, r) for r in Rs]
    sq = dj(Sm, eqt)

    def body(*refs):
        ea_ref = refs[0]
        xp_refs = refs[1:1 + nx]
        (waq_ref, ba_ref, wb_ref, bb_ref) = refs[1 + nx:5 + nx]
        rq_refs = refs[5 + nx:5 + 2 * nx]
        (sq_ref, ex_ref, out_ref) = refs[5 + 2 * nx:]
        dot = functools.partial(jnp.dot, preferred_element_type=f32)
        eap = ea_ref[...]
        xpv = [r[...] for r in xp_refs]
        acc = ex_ref[...] + jnp.zeros((PR, 128), f32)
        for q in range(8):
            h = jnp.maximum(
                dot(eap, waq_ref[pl.ds(q * 128, 128), :]) + ba_ref[...], 0.0)
            we = dot(h, wb_ref[...]) + bb_ref[...]
            xt = dot(xpv[0], rq_refs[0][pl.ds(q * 128, 128), :])
            for t in range(1, nx):
                xt = xt + dot(xpv[t], rq_refs[t][pl.ds(q * 128, 128), :])
            acc = acc + dot(we * xt, sq_ref[:, pl.ds(q * 128, 128)])
        out_ref[...] = acc

    full = lambda shape: pl.BlockSpec(shape, lambda i: (0, 0))
    return pl.pallas_call(
        body,
        grid=(G,),
        in_specs=[pl.BlockSpec((PR, 128), lambda i: (i, 0))] * (1 + nx)
        + [full((8 * 128, Ha)), full((1, Ha)), full((Ha, Ha)), full((1, Ha))]
        + [full((8 * 128, Ha))] * nx
        + [full((Ha, 8 * 128)), full((1, 128))],
        out_specs=pl.BlockSpec((PR, 128), lambda i: (i, 0)),
        out_shape=jax.ShapeDtypeStruct((EP, 128), jnp.float32),
    )(ea_p, *xps, waq, ba, Wb, bb, *rq, sq, exp)


NP8 = N // 8
PPAD = NPAD // 8


def _post(parts, invp_in, xps_cur, roots, bias, g, be, c_out, with_cnt):
    f32 = jnp.float32
    dj = functools.partial(jnp.dot, preferred_element_type=f32)
    eqs, eqt = _eq_consts()
    nt = len(xps_cur)
    csel = jnp.asarray(np.eye(DOUT, c_out, dtype=np.float32))
    ec = dj(eqs, csel)
    erq = [dj(eqs, r) for r in roots]
    poq = jnp.concatenate(
        [eqt[:c_out, q * 128:(q + 1) * 128] for q in range(8)], axis=0)
    pp = parts.reshape(NC * PPAD, 128)
    ins = [pp] + ([] if with_cnt else [invp_in]) + list(xps_cur) \
        + [ec] + erq + [poq, bias, g, be]
    outs = [jax.ShapeDtypeStruct((NP8, 128), jnp.float32)]
    if with_cnt:
        e8 = dj(eqs, jnp.asarray(
            np.eye(DOUT, 1, k=-c_out, dtype=np.float32)))
        ones8 = np.zeros((8, 128), np.float32)
        for q in range(8):
            ones8[q, q * DOUT:(q + 1) * DOUT] = 1.0
        ins += [e8, jnp.asarray(ones8)]
        outs.append(jax.ShapeDtypeStruct((NP8, 128), jnp.float32))

    def body(*refs):
        i = 0
        pp_ref = refs[i]; i += 1
        if not with_cnt:
            invp_ref = refs[i]; i += 1
        xp_refs = refs[i:i + nt]; i += nt
        ec_ref = refs[i]; i += 1
        erq_refs = refs[i:i + nt]; i += nt
        poq_ref, bias_ref, g_ref, be_ref = refs[i:i + 4]; i += 4
        if with_cnt:
            e8_ref, ones_ref = refs[i:i + 2]; i += 2
            out_ref, invp_out = refs[i:i + 2]
        else:
            out_ref = refs[i]
        acc = pp_ref[0:NP8, :] + pp_ref[PPAD:PPAD + NP8, :]
        if not with_cnt:
            acc = acc * invp_ref[...]
        xpv = [r[...] for r in xp_refs]
        hs, invs = [], []
        su = jnp.zeros((1, c_out), f32)
        ssq = jnp.zeros((1, c_out), f32)
        for q in range(8):
            hq = dj(acc, ec_ref[pl.ds(q * 128, 128), :])
            if with_cnt:
                cnt = dj(acc, e8_ref[pl.ds(q * 128, 128), :])
                inv = 1.0 / jnp.maximum(cnt, 1.0)
                invs.append(inv)
                hq = hq * inv
            for t in range(nt):
                hq = hq + dj(xpv[t], erq_refs[t][pl.ds(q * 128, 128), :])
            hq = hq + bias_ref[...]
            hs.append(hq)
            su = su + jnp.sum(hq, axis=0, keepdims=True)
            ssq = ssq + jnp.sum(hq * hq, axis=0, keepdims=True)
        mu = su * (1.0 / N)
        var = ssq * (1.0 / N) - mu * mu
        scale = g_ref[...] * lax.rsqrt(var + EPS)
        shift = be_ref[...] - mu * scale
        out = jnp.zeros((NP8, 128), f32)
        for q in range(8):
            y = jnp.maximum(hs[q] * scale + shift, 0.0)
            out = out + dj(y, poq_ref[pl.ds(q * c_out, c_out), :])
        out_ref[...] = out
        if with_cnt:
            ip = jnp.zeros((NP8, 128), f32)
            for q in range(8):
                ip = ip + dj(invs[q], ones_ref[pl.ds(q, 1), :])
            invp_out[...] = ip

    res = pl.pallas_call(body, out_shape=outs)(*ins)
    return res if with_cnt else res[0]


def _mk_RS(c_in, c_out):
    ha = c_in * c_out
    fp = 32 if c_in == 32 else DOUT
    rm = np.zeros((fp, ha), np.float32)
    sm = np.zeros((ha, DOUT), np.float32)
    for i in range(c_in):
        for o in range(c_out):
            rm[i, i * c_out + o] = 1.0
            sm[i * c_out + o, o] = 1.0
    return jnp.asarray(rm), jnp.asarray(sm)


def kernel(x, edge_index, edge_attr, W1a, b1a, W1b, b1b, root1, bias1, g1, be1,
           W2a, b2a, W2b, b2b, root2, bias2, g2, be2,
           W3a, b3a, W3b, b3b, root3, bias3, g3, be3):
    src = edge_index[0].astype(jnp.int32).reshape(NW, C, K)
    dst = edge_index[1].astype(jnp.int32).reshape(NW, C, K)
    zinit = jnp.zeros((NPAD, DOUT), jnp.float32)
    ea_p = edge_attr.reshape(EP, 128)

    r1, s1 = _mk_RS(32, 8)
    r2, s2 = _mk_RS(8, 4)
    r3, s3 = _mk_RS(4, 16)
    ex1 = np.zeros((1, DOUT), np.float32)
    ex1[0, 8] = 1.0
    ex1 = jnp.asarray(ex1)
    ex0 = jnp.zeros((1, DOUT), jnp.float32)

    def row(v):
        return v.reshape(1, -1)

    xa16, xb16 = x[:, :16], x[:, 16:]
    xa, xb = _gather16([xa16, xb16], src)
    msg = _msg(ea_p, [xa, xb], W1a, row(b1a), W1b, row(b1b),
               [r1[:16], r1[16:]], s1, ex1)
    parts = _scatter(msg, dst, zinit)
    h1p, invp = _post(parts, None,
                      [xa16.reshape(NP8, 128), xb16.reshape(NP8, 128)],
                      [root1[:16], root1[16:]], row(bias1),
                      row(g1), row(be1), 8, True)
    h1 = h1p.reshape(N, DOUT)

    xj, = _gather16([h1], src)
    msg = _msg(ea_p, [xj], W2a, row(b2a), W2b, row(b2b), [r2], s2, ex0)
    parts = _scatter(msg, dst, zinit)
    h2p = _post(parts, invp, [h1p], [jnp.pad(root2, ((0, 8), (0, 0)))],
                row(bias2), row(g2), row(be2), 4, False)
    h2 = h2p.reshape(N, DOUT)

    xj, = _gather16([h2], src)
    msg = _msg(ea_p, [xj], W3a, row(b3a), W3b, row(b3b), [r3], s3, ex0)
    parts = _scatter(msg, dst, zinit)
    h3p = _post(parts, invp, [h2p], [jnp.pad(root3, ((0, 12), (0, 0)))],
                row(bias3), row(g3), row(be3), 16, False)
    return h3p.reshape(N, DOUT)


# speedup vs baseline: 1.3608x; 1.0483x over previous
import functools

import numpy as np
import jax
import jax.numpy as jnp
from jax import lax
from jax.experimental import pallas as pl
from jax.experimental.pallas import tpu as pltpu
from jax.experimental.pallas import tpu_sc as plsc

N = 10000
E = 160000
NC, NS = 2, 16
NW = NC * NS
K = 125
C = E // (NW * K)
PW = C * K
CPB = 8
NB = C // CPB
NPAD = 10240
RP = NPAD // NS
DOUT = 16
EP = E // 8
EPS = 1e-5
_MESH = dict(core_axis_name="c", subcore_axis_name="s")


def _gather16(tables, idx_w):
    nt = len(tables)
    mesh = plsc.VectorSubcoreMesh(**_MESH)

    @functools.partial(
        pl.kernel,
        out_type=[jax.ShapeDtypeStruct((E, DOUT), jnp.float32)] * nt,
        mesh=mesh,
        compiler_params=pltpu.CompilerParams(use_tc_tiling_on_sc=False),
        scratch_types=[pltpu.VMEM((C, K), jnp.int32)]
        + [pltpu.VMEM((CPB * K, DOUT), jnp.float32)] * nt
        + [pltpu.SemaphoreType.DMA],
    )
    def gk(*refs):
        tabs = refs[:nt]
        idx_hbm = refs[nt]
        outs = refs[nt + 1:2 * nt + 1]
        idx_v = refs[2 * nt + 1]
        bufs = refs[2 * nt + 2:3 * nt + 2]
        sem = refs[3 * nt + 2]
        wid = lax.axis_index("s") * NC + lax.axis_index("c")
        base = wid * PW
        pltpu.sync_copy(idx_hbm.at[0, wid], idx_v)

        @pl.loop(0, NB)
        def _(cc):
            cps = []
            for t in range(CPB):
                for tab, buf in zip(tabs, bufs):
                    cps.append(pltpu.async_copy(
                        tab.at[idx_v.at[cc * CPB + t]],
                        buf.at[pl.ds(t * K, K)], sem))
            for cp in cps:
                cp.wait()
            for buf, out in zip(bufs, outs):
                pltpu.sync_copy(
                    buf, out.at[pl.ds(base + cc * (CPB * K), CPB * K)])

    res = gk(*tables, idx_w)
    if not isinstance(res, (list, tuple)):
        res = [res]
    return [r.reshape(EP, 128) for r in res]


def _scatter(msg_p, idx_w, zinit):
    mesh = plsc.VectorSubcoreMesh(**_MESH)

    @functools.partial(
        pl.kernel,
        out_type=jax.ShapeDtypeStruct((NC, NPAD, DOUT), jnp.float32),
        mesh=mesh,
        compiler_params=pltpu.CompilerParams(use_tc_tiling_on_sc=False),
        scratch_types=[
            pltpu.VMEM((C, K), jnp.int32),
            pltpu.VMEM((PW, DOUT), jnp.float32),
            pltpu.VMEM_SHARED((NPAD, DOUT), jnp.float32),
            pltpu.SemaphoreType.DMA,
        ],
    )
    def sk(msg_hbm, idx_hbm, zero_hbm, out_hbm, idx_v, msg_v, acc_sh, sem):
        cid = lax.axis_index("c")
        sid = lax.axis_index("s")
        wid = sid * NC + cid
        row0 = sid * RP
        pltpu.sync_copy(zero_hbm.at[pl.ds(row0, RP)], acc_sh.at[pl.ds(row0, RP)])
        plsc.subcore_barrier()
        pltpu.sync_copy(msg_hbm.at[pl.ds(wid * PW, PW)], msg_v)
        pltpu.sync_copy(idx_hbm.at[1, wid], idx_v)

        @pl.loop(0, NB)
        def _(cc):
            cps = []
            for t in range(CPB):
                j = cc * CPB + t
                cps.append(pltpu.async_copy(
                    msg_v.at[pl.ds(j * K, K)], acc_sh.at[idx_v.at[j]], sem,
                    add=True))
            for cp in cps:
                cp.wait()

        plsc.subcore_barrier()
        pltpu.sync_copy(acc_sh.at[pl.ds(row0, RP)], out_hbm.at[cid, pl.ds(row0, RP)])

    return sk(msg_p.reshape(E, DOUT), idx_w, zinit)


def _eq_consts():
    eqs_np = np.zeros((8 * 128, DOUT), np.float32)
    for q in range(8):
        for c in range(DOUT):
            eqs_np[q * 128 + q * DOUT + c, c] = 1.0
    eqt_np = np.concatenate(
        [eqs_np[q * 128:(q + 1) * 128].T for q in range(8)], axis=1)
    return jnp.asarray(eqs_np), jnp.asarray(eqt_np)


def _msg(ea_p, xps, Wa, ba, Wb, bb, Rs, Sm, extra, block_e=16000):
    G = E // block_e
    PR = block_e // 8
    Ha = Wa.shape[1]
    nx = len(xps)
    eqs, eqt = _eq_consts()
    f32 = jnp.float32
    dj = functools.partial(jnp.dot, preferred_element_type=f32)
    exp = jnp.tile(extra, (1, 8))

    if Ha <= 64:
        HW = 8 * Ha
        waw = jnp.concatenate(
            [dj(eqs[q * 128:(q + 1) * 128], Wa) for q in range(8)], axis=1)
        rqw = [jnp.concatenate(
            [dj(eqs[q * 128:(q + 1) * 128], r) for q in range(8)], axis=1)
            for r in Rs]
        wbd = jnp.kron(jnp.eye(8, dtype=f32), Wb)
        sqw = jnp.concatenate(
            [dj(Sm, eqt[:, q * 128:(q + 1) * 128]) for q in range(8)], axis=0)
        baw = jnp.tile(ba, (1, 8))
        dsq = jnp.dot(jnp.diag(bb[0]), Sm, preferred_element_type=f32)
        dsqw = jnp.concatenate(
            [dj(dsq, eqt[:, q * 128:(q + 1) * 128]) for q in range(8)], axis=0)

        def body(*refs):
            ea_ref = refs[0]
            xp_refs = refs[1:1 + nx]
            (waw_ref, baw_ref, wbd_ref) = refs[1 + nx:4 + nx]
            rq_refs = refs[4 + nx:4 + 2 * nx]
            (sqw_ref, dsq_ref, ex_ref, out_ref) = refs[4 + 2 * nx:]
            eap = ea_ref[...]
            h = jnp.maximum(dj(eap, waw_ref[...]) + baw_ref[...], 0.0)
            we = dj(h, wbd_ref[...])
            xt = dj(xp_refs[0][...], rq_refs[0][...])
            for t in range(1, nx):
                xt = xt + dj(xp_refs[t][...], rq_refs[t][...])
            out_ref[...] = (ex_ref[...] + dj(we * xt, sqw_ref[...])
                            + dj(xt, dsq_ref[...]))

        full = lambda shape: pl.BlockSpec(shape, lambda i: (0, 0))
        return pl.pallas_call(
            body,
            grid=(G,),
            in_specs=[pl.BlockSpec((PR, 128), lambda i: (i, 0))] * (1 + nx)
            + [full((128, HW)), full((1, HW)), full((HW, HW))]
            + [full((128, HW))] * nx
            + [full((HW, 128)), full((HW, 128)), full((1, 128))],
            out_specs=pl.BlockSpec((PR, 128), lambda i: (i, 0)),
            out_shape=jax.ShapeDtypeStruct((EP, 128), jnp.float32),
        )(ea_p, *xps, waw, baw, wbd, *rqw, sqw, dsqw, exp)

    waq = dj(eqs, Wa)
    rq = [dj(eqs, r) for r in Rs]
    sq = dj(Sm, eqt)

    def body(*refs):
        ea_ref = refs[0]
        xp_refs = refs[1:1 + nx]
        (waq_ref, ba_ref, wb_ref, bb_ref) = refs[1 + nx:5 + nx]
        rq_refs = refs[5 + nx:5 + 2 * nx]
        (sq_ref, ex_ref, out_ref) = refs[5 + 2 * nx:]
        dot = functools.partial(jnp.dot, preferred_element_type=f32)
        eap = ea_ref[...]
        xpv = [r[...] for r in xp_refs]
        acc = ex_ref[...] + jnp.zeros((PR, 128), f32)
        for q in range(8):
            h = jnp.maximum(
                dot(eap, waq_ref[pl.ds(q * 128, 128), :]) + ba_ref[...], 0.0)
            we = dot(h, wb_ref[...]) + bb_ref[...]
            xt = dot(xpv[0], rq_refs[0][pl.ds(q * 128, 128), :])
            for t in range(1, nx):
                xt = xt + dot(xpv[t], rq_refs[t][pl.ds(q * 128, 128), :])
            acc = acc + dot(we * xt, sq_ref[:, pl.ds(q * 128, 128)])
        out_ref[...] = acc

    full = lambda shape: pl.BlockSpec(shape, lambda i: (0, 0))
    return pl.pallas_call(
        body,
        grid=(G,),
        in_specs=[pl.BlockSpec((PR, 128), lambda i: (i, 0))] * (1 + nx)
        + [full((8 * 128, Ha)), full((1, Ha)), full((Ha, Ha)), full((1, Ha))]
        + [full((8 * 128, Ha))] * nx
        + [full((Ha, 8 * 128)), full((1, 128))],
        out_specs=pl.BlockSpec((PR, 128), lambda i: (i, 0)),
        out_shape=jax.ShapeDtypeStruct((EP, 128), jnp.float32),
    )(ea_p, *xps, waq, ba, Wb, bb, *rq, sq, exp)


NP8 = N // 8
PPAD = NPAD // 8


def _post(parts, invp_in, xps_cur, roots, bias, g, be, c_out, with_cnt):
    f32 = jnp.float32
    dj = functools.partial(jnp.dot, preferred_element_type=f32)
    eqs, eqt = _eq_consts()
    nt = len(xps_cur)
    csel = jnp.asarray(np.eye(DOUT, c_out, dtype=np.float32))
    ec = dj(eqs, csel)
    erq = [dj(eqs, r) for r in roots]
    poq = jnp.concatenate(
        [eqt[:c_out, q * 128:(q + 1) * 128] for q in range(8)], axis=0)
    pp = parts.reshape(NC * PPAD, 128)
    ins = [pp] + ([] if with_cnt else [invp_in]) + list(xps_cur) \
        + [ec] + erq + [poq, bias, g, be]
    outs = [jax.ShapeDtypeStruct((NP8, 128), jnp.float32)]
    if with_cnt:
        e8 = dj(eqs, jnp.asarray(
            np.eye(DOUT, 1, k=-c_out, dtype=np.float32)))
        ones8 = np.zeros((8, 128), np.float32)
        for q in range(8):
            ones8[q, q * DOUT:(q + 1) * DOUT] = 1.0
        ins += [e8, jnp.asarray(ones8)]
        outs.append(jax.ShapeDtypeStruct((NP8, 128), jnp.float32))

    def body(*refs):
        i = 0
        pp_ref = refs[i]; i += 1
        if not with_cnt:
            invp_ref = refs[i]; i += 1
        xp_refs = refs[i:i + nt]; i += nt
        ec_ref = refs[i]; i += 1
        erq_refs = refs[i:i + nt]; i += nt
        poq_ref, bias_ref, g_ref, be_ref = refs[i:i + 4]; i += 4
        if with_cnt:
            e8_ref, ones_ref = refs[i:i + 2]; i += 2
            out_ref, invp_out = refs[i:i + 2]
        else:
            out_ref = refs[i]
        acc = pp_ref[0:NP8, :] + pp_ref[PPAD:PPAD + NP8, :]
        if not with_cnt:
            acc = acc * invp_ref[...]
        xpv = [r[...] for r in xp_refs]
        hs, invs = [], []
        su = jnp.zeros((1, c_out), f32)
        ssq = jnp.zeros((1, c_out), f32)
        for q in range(8):
            hq = dj(acc, ec_ref[pl.ds(q * 128, 128), :])
            if with_cnt:
                cnt = dj(acc, e8_ref[pl.ds(q * 128, 128), :])
                inv = 1.0 / jnp.maximum(cnt, 1.0)
                invs.append(inv)
                hq = hq * inv
            for t in range(nt):
                hq = hq + dj(xpv[t], erq_refs[t][pl.ds(q * 128, 128), :])
            hq = hq + bias_ref[...]
            hs.append(hq)
            su = su + jnp.sum(hq, axis=0, keepdims=True)
            ssq = ssq + jnp.sum(hq * hq, axis=0, keepdims=True)
        mu = su * (1.0 / N)
        var = ssq * (1.0 / N) - mu * mu
        scale = g_ref[...] * lax.rsqrt(var + EPS)
        shift = be_ref[...] - mu * scale
        out = jnp.zeros((NP8, 128), f32)
        for q in range(8):
            y = jnp.maximum(hs[q] * scale + shift, 0.0)
            out = out + dj(y, poq_ref[pl.ds(q * c_out, c_out), :])
        out_ref[...] = out
        if with_cnt:
            ip = jnp.zeros((NP8, 128), f32)
            for q in range(8):
                ip = ip + dj(invs[q], ones_ref[pl.ds(q, 1), :])
            invp_out[...] = ip

    res = pl.pallas_call(body, out_shape=outs)(*ins)
    return res if with_cnt else res[0]


def _mk_RS(c_in, c_out):
    ha = c_in * c_out
    fp = 32 if c_in == 32 else DOUT
    rm = np.zeros((fp, ha), np.float32)
    sm = np.zeros((ha, DOUT), np.float32)
    for i in range(c_in):
        for o in range(c_out):
            rm[i, i * c_out + o] = 1.0
            sm[i * c_out + o, o] = 1.0
    return jnp.asarray(rm), jnp.asarray(sm)


def kernel(x, edge_index, edge_attr, W1a, b1a, W1b, b1b, root1, bias1, g1, be1,
           W2a, b2a, W2b, b2b, root2, bias2, g2, be2,
           W3a, b3a, W3b, b3b, root3, bias3, g3, be3):
    idx = edge_index.astype(jnp.int32).reshape(2, NW, C, K)
    zinit = jnp.zeros((NPAD, DOUT), jnp.float32)
    ea_p = edge_attr.reshape(EP, 128)

    r1, s1 = _mk_RS(32, 8)
    r2, s2 = _mk_RS(8, 4)
    r3, s3 = _mk_RS(4, 16)
    ex1 = np.zeros((1, DOUT), np.float32)
    ex1[0, 8] = 1.0
    ex1 = jnp.asarray(ex1)
    ex0 = jnp.zeros((1, DOUT), jnp.float32)

    def row(v):
        return v.reshape(1, -1)

    xa16, xb16 = x[:, :16], x[:, 16:]
    xa, xb = _gather16([xa16, xb16], idx)
    msg = _msg(ea_p, [xa, xb], W1a, row(b1a), W1b, row(b1b),
               [r1[:16], r1[16:]], s1, ex1)
    parts = _scatter(msg, idx, zinit)
    h1p, invp = _post(parts, None,
                      [xa16.reshape(NP8, 128), xb16.reshape(NP8, 128)],
                      [root1[:16], root1[16:]], row(bias1),
                      row(g1), row(be1), 8, True)
    h1 = h1p.reshape(N, DOUT)

    xj, = _gather16([h1], idx)
    msg = _msg(ea_p, [xj], W2a, row(b2a), W2b, row(b2b), [r2], s2, ex0)
    parts = _scatter(msg, idx, zinit)
    h2p = _post(parts, invp, [h1p], [jnp.pad(root2, ((0, 8), (0, 0)))],
                row(bias2), row(g2), row(be2), 4, False)
    h2 = h2p.reshape(N, DOUT)

    xj, = _gather16([h2], idx)
    msg = _msg(ea_p, [xj], W3a, row(b3a), W3b, row(b3b), [r3], s3, ex0)
    parts = _scatter(msg, idx, zinit)
    h3p = _post(parts, invp, [h2p], [jnp.pad(root3, ((0, 12), (0, 0)))],
                row(bias3), row(g3), row(be3), 16, False)
    return h3p.reshape(N, DOUT)


# speedup vs baseline: 1.3648x; 1.0029x over previous
import functools

import numpy as np
import jax
import jax.numpy as jnp
from jax import lax
from jax.experimental import pallas as pl
from jax.experimental.pallas import tpu as pltpu
from jax.experimental.pallas import tpu_sc as plsc

N = 10000
E = 160000
NC, NS = 2, 16
NW = NC * NS
K = 125
C = E // (NW * K)
PW = C * K
CPB = 8
NB = C // CPB
NPAD = 10240
RP = NPAD // NS
DOUT = 16
EP = E // 8
EPS = 1e-5
_MESH = dict(core_axis_name="c", subcore_axis_name="s")


def _gather16(tables, idx_w):
    nt = len(tables)
    mesh = plsc.VectorSubcoreMesh(**_MESH)

    @functools.partial(
        pl.kernel,
        out_type=[jax.ShapeDtypeStruct((E, DOUT), jnp.float32)] * nt,
        mesh=mesh,
        compiler_params=pltpu.CompilerParams(use_tc_tiling_on_sc=False),
        scratch_types=[pltpu.VMEM((C, K), jnp.int32)]
        + [pltpu.VMEM((CPB * K, DOUT), jnp.float32)] * nt
        + [pltpu.SemaphoreType.DMA],
    )
    def gk(*refs):
        tabs = refs[:nt]
        idx_hbm = refs[nt]
        outs = refs[nt + 1:2 * nt + 1]
        idx_v = refs[2 * nt + 1]
        bufs = refs[2 * nt + 2:3 * nt + 2]
        sem = refs[3 * nt + 2]
        wid = lax.axis_index("s") * NC + lax.axis_index("c")
        base = wid * PW
        pltpu.sync_copy(idx_hbm.at[0, wid], idx_v)

        @pl.loop(0, NB)
        def _(cc):
            cps = []
            for t in range(CPB):
                for tab, buf in zip(tabs, bufs):
                    cps.append(pltpu.async_copy(
                        tab.at[idx_v.at[cc * CPB + t]],
                        buf.at[pl.ds(t * K, K)], sem))
            for cp in cps:
                cp.wait()
            for buf, out in zip(bufs, outs):
                pltpu.sync_copy(
                    buf, out.at[pl.ds(base + cc * (CPB * K), CPB * K)])

    res = gk(*tables, idx_w)
    if not isinstance(res, (list, tuple)):
        res = [res]
    return [r.reshape(EP, 128) for r in res]


def _scatter(msg_p, idx_w, zinit):
    mesh = plsc.VectorSubcoreMesh(**_MESH)

    @functools.partial(
        pl.kernel,
        out_type=jax.ShapeDtypeStruct((NC, NPAD, DOUT), jnp.float32),
        mesh=mesh,
        compiler_params=pltpu.CompilerParams(use_tc_tiling_on_sc=False),
        scratch_types=[
            pltpu.VMEM((C, K), jnp.int32),
            pltpu.VMEM((PW, DOUT), jnp.float32),
            pltpu.VMEM_SHARED((NPAD, DOUT), jnp.float32),
            pltpu.SemaphoreType.DMA,
        ],
    )
    def sk(msg_hbm, idx_hbm, zero_hbm, out_hbm, idx_v, msg_v, acc_sh, sem):
        cid = lax.axis_index("c")
        sid = lax.axis_index("s")
        wid = sid * NC + cid
        row0 = sid * RP
        pltpu.sync_copy(zero_hbm.at[pl.ds(row0, RP)], acc_sh.at[pl.ds(row0, RP)])
        plsc.subcore_barrier()
        pltpu.sync_copy(msg_hbm.at[pl.ds(wid * PW, PW)], msg_v)
        pltpu.sync_copy(idx_hbm.at[1, wid], idx_v)

        @pl.loop(0, NB)
        def _(cc):
            cps = []
            for t in range(CPB):
                j = cc * CPB + t
                cps.append(pltpu.async_copy(
                    msg_v.at[pl.ds(j * K, K)], acc_sh.at[idx_v.at[j]], sem,
                    add=True))
            for cp in cps:
                cp.wait()

        plsc.subcore_barrier()
        pltpu.sync_copy(acc_sh.at[pl.ds(row0, RP)], out_hbm.at[cid, pl.ds(row0, RP)])

    return sk(msg_p.reshape(E, DOUT), idx_w, zinit)


def _eq_consts():
    eqs_np = np.zeros((8 * 128, DOUT), np.float32)
    for q in range(8):
        for c in range(DOUT):
            eqs_np[q * 128 + q * DOUT + c, c] = 1.0
    eqt_np = np.concatenate(
        [eqs_np[q * 128:(q + 1) * 128].T for q in range(8)], axis=1)
    return jnp.asarray(eqs_np), jnp.asarray(eqt_np)


def _msg(ea_p, xps, Wa, ba, Wb, bb, Rs, Sm, extra, block_e=16000):
    G = E // block_e
    PR = block_e // 8
    Ha = Wa.shape[1]
    nx = len(xps)
    eqs, eqt = _eq_consts()
    f32 = jnp.float32
    dj = functools.partial(jnp.dot, preferred_element_type=f32)
    exp = jnp.tile(extra, (1, 8))

    if Ha <= 64:
        HW = 8 * Ha
        waw = jnp.concatenate(
            [dj(eqs[q * 128:(q + 1) * 128], Wa) for q in range(8)], axis=1)
        rqw = [jnp.concatenate(
            [dj(eqs[q * 128:(q + 1) * 128], r) for q in range(8)], axis=1)
            for r in Rs]
        wbd = jnp.kron(jnp.eye(8, dtype=f32), Wb).astype(jnp.bfloat16)
        sqw = jnp.concatenate(
            [dj(Sm, eqt[:, q * 128:(q + 1) * 128]) for q in range(8)], axis=0)
        baw = jnp.tile(ba, (1, 8))
        dsq = jnp.dot(jnp.diag(bb[0]), Sm, preferred_element_type=f32)
        dsqw = jnp.concatenate(
            [dj(dsq, eqt[:, q * 128:(q + 1) * 128]) for q in range(8)], axis=0)

        def body(*refs):
            ea_ref = refs[0]
            xp_refs = refs[1:1 + nx]
            (waw_ref, baw_ref, wbd_ref) = refs[1 + nx:4 + nx]
            rq_refs = refs[4 + nx:4 + 2 * nx]
            (sqw_ref, dsq_ref, ex_ref, out_ref) = refs[4 + 2 * nx:]
            eap = ea_ref[...]
            h = jnp.maximum(dj(eap, waw_ref[...]) + baw_ref[...], 0.0)
            we = jnp.dot(h.astype(jnp.bfloat16), wbd_ref[...],
                         preferred_element_type=jnp.float32)
            xt = dj(xp_refs[0][...], rq_refs[0][...])
            for t in range(1, nx):
                xt = xt + dj(xp_refs[t][...], rq_refs[t][...])
            out_ref[...] = (ex_ref[...] + dj(we * xt, sqw_ref[...])
                            + dj(xt, dsq_ref[...]))

        full = lambda shape: pl.BlockSpec(shape, lambda i: (0, 0))
        return pl.pallas_call(
            body,
            grid=(G,),
            in_specs=[pl.BlockSpec((PR, 128), lambda i: (i, 0))] * (1 + nx)
            + [full((128, HW)), full((1, HW)), full((HW, HW))]
            + [full((128, HW))] * nx
            + [full((HW, 128)), full((HW, 128)), full((1, 128))],
            out_specs=pl.BlockSpec((PR, 128), lambda i: (i, 0)),
            out_shape=jax.ShapeDtypeStruct((EP, 128), jnp.float32),
        )(ea_p, *xps, waw, baw, wbd, *rqw, sqw, dsqw, exp)

    waq = dj(eqs, Wa)
    rq = [dj(eqs, r) for r in Rs]
    sq = dj(Sm, eqt)

    def body(*refs):
        ea_ref = refs[0]
        xp_refs = refs[1:1 + nx]
        (waq_ref, ba_ref, wb_ref, bb_ref) = refs[1 + nx:5 + nx]
        rq_refs = refs[5 + nx:5 + 2 * nx]
        (sq_ref, ex_ref, out_ref) = refs[5 + 2 * nx:]
        dot = functools.partial(jnp.dot, preferred_element_type=f32)
        eap = ea_ref[...]
        xpv = [r[...] for r in xp_refs]
        acc = ex_ref[...] + jnp.zeros((PR, 128), f32)
        for q in range(8):
            h = jnp.maximum(
                dot(eap, waq_ref[pl.ds(q * 128, 128), :]) + ba_ref[...], 0.0)
            we = jnp.dot(h.astype(jnp.bfloat16), wb_ref[...],
                         preferred_element_type=jnp.float32) + bb_ref[...]
            xt = dot(xpv[0], rq_refs[0][pl.ds(q * 128, 128), :])
            for t in range(1, nx):
                xt = xt + dot(xpv[t], rq_refs[t][pl.ds(q * 128, 128), :])
            acc = acc + dot(we * xt, sq_ref[:, pl.ds(q * 128, 128)])
        out_ref[...] = acc

    full = lambda shape: pl.BlockSpec(shape, lambda i: (0, 0))
    return pl.pallas_call(
        body,
        grid=(G,),
        in_specs=[pl.BlockSpec((PR, 128), lambda i: (i, 0))] * (1 + nx)
        + [full((8 * 128, Ha)), full((1, Ha)), full((Ha, Ha)), full((1, Ha))]
        + [full((8 * 128, Ha))] * nx
        + [full((Ha, 8 * 128)), full((1, 128))],
        out_specs=pl.BlockSpec((PR, 128), lambda i: (i, 0)),
        out_shape=jax.ShapeDtypeStruct((EP, 128), jnp.float32),
    )(ea_p, *xps, waq, ba, Wb.astype(jnp.bfloat16), bb, *rq, sq, exp)


NP8 = N // 8
PPAD = NPAD // 8


def _post(parts, invp_in, xps_cur, roots, bias, g, be, c_out, with_cnt):
    f32 = jnp.float32
    dj = functools.partial(jnp.dot, preferred_element_type=f32)
    eqs, eqt = _eq_consts()
    nt = len(xps_cur)
    csel = jnp.asarray(np.eye(DOUT, c_out, dtype=np.float32))
    ec = dj(eqs, csel)
    erq = [dj(eqs, r) for r in roots]
    poq = jnp.concatenate(
        [eqt[:c_out, q * 128:(q + 1) * 128] for q in range(8)], axis=0)
    pp = parts.reshape(NC * PPAD, 128)
    ins = [pp] + ([] if with_cnt else [invp_in]) + list(xps_cur) \
        + [ec] + erq + [poq, bias, g, be]
    outs = [jax.ShapeDtypeStruct((NP8, 128), jnp.float32)]
    if with_cnt:
        e8 = dj(eqs, jnp.asarray(
            np.eye(DOUT, 1, k=-c_out, dtype=np.float32)))
        ones8 = np.zeros((8, 128), np.float32)
        for q in range(8):
            ones8[q, q * DOUT:(q + 1) * DOUT] = 1.0
        ins += [e8, jnp.asarray(ones8)]
        outs.append(jax.ShapeDtypeStruct((NP8, 128), jnp.float32))

    def body(*refs):
        i = 0
        pp_ref = refs[i]; i += 1
        if not with_cnt:
            invp_ref = refs[i]; i += 1
        xp_refs = refs[i:i + nt]; i += nt
        ec_ref = refs[i]; i += 1
        erq_refs = refs[i:i + nt]; i += nt
        poq_ref, bias_ref, g_ref, be_ref = refs[i:i + 4]; i += 4
        if with_cnt:
            e8_ref, ones_ref = refs[i:i + 2]; i += 2
            out_ref, invp_out = refs[i:i + 2]
        else:
            out_ref = refs[i]
        acc = pp_ref[0:NP8, :] + pp_ref[PPAD:PPAD + NP8, :]
        if not with_cnt:
            acc = acc * invp_ref[...]
        xpv = [r[...] for r in xp_refs]
        hs, invs = [], []
        su = jnp.zeros((1, c_out), f32)
        ssq = jnp.zeros((1, c_out), f32)
        for q in range(8):
            hq = dj(acc, ec_ref[pl.ds(q * 128, 128), :])
            if with_cnt:
                cnt = dj(acc, e8_ref[pl.ds(q * 128, 128), :])
                inv = 1.0 / jnp.maximum(cnt, 1.0)
                invs.append(inv)
                hq = hq * inv
            for t in range(nt):
                hq = hq + dj(xpv[t], erq_refs[t][pl.ds(q * 128, 128), :])
            hq = hq + bias_ref[...]
            hs.append(hq)
            su = su + jnp.sum(hq, axis=0, keepdims=True)
            ssq = ssq + jnp.sum(hq * hq, axis=0, keepdims=True)
        mu = su * (1.0 / N)
        var = ssq * (1.0 / N) - mu * mu
        scale = g_ref[...] * lax.rsqrt(var + EPS)
        shift = be_ref[...] - mu * scale
        out = jnp.zeros((NP8, 128), f32)
        for q in range(8):
            y = jnp.maximum(hs[q] * scale + shift, 0.0)
            out = out + dj(y, poq_ref[pl.ds(q * c_out, c_out), :])
        out_ref[...] = out
        if with_cnt:
            ip = jnp.zeros((NP8, 128), f32)
            for q in range(8):
                ip = ip + dj(invs[q], ones_ref[pl.ds(q, 1), :])
            invp_out[...] = ip

    res = pl.pallas_call(body, out_shape=outs)(*ins)
    return res if with_cnt else res[0]


def _mk_RS(c_in, c_out):
    ha = c_in * c_out
    fp = 32 if c_in == 32 else DOUT
    rm = np.zeros((fp, ha), np.float32)
    sm = np.zeros((ha, DOUT), np.float32)
    for i in range(c_in):
        for o in range(c_out):
            rm[i, i * c_out + o] = 1.0
            sm[i * c_out + o, o] = 1.0
    return jnp.asarray(rm), jnp.asarray(sm)


def kernel(x, edge_index, edge_attr, W1a, b1a, W1b, b1b, root1, bias1, g1, be1,
           W2a, b2a, W2b, b2b, root2, bias2, g2, be2,
           W3a, b3a, W3b, b3b, root3, bias3, g3, be3):
    idx = edge_index.astype(jnp.int32).reshape(2, NW, C, K)
    zinit = jnp.zeros((NPAD, DOUT), jnp.float32)
    ea_p = edge_attr.reshape(EP, 128)

    r1, s1 = _mk_RS(32, 8)
    r2, s2 = _mk_RS(8, 4)
    r3, s3 = _mk_RS(4, 16)
    ex1 = np.zeros((1, DOUT), np.float32)
    ex1[0, 8] = 1.0
    ex1 = jnp.asarray(ex1)
    ex0 = jnp.zeros((1, DOUT), jnp.float32)

    def row(v):
        return v.reshape(1, -1)

    xa16, xb16 = x[:, :16], x[:, 16:]
    xa, xb = _gather16([xa16, xb16], idx)
    msg = _msg(ea_p, [xa, xb], W1a, row(b1a), W1b, row(b1b),
               [r1[:16], r1[16:]], s1, ex1)
    parts = _scatter(msg, idx, zinit)
    h1p, invp = _post(parts, None,
                      [xa16.reshape(NP8, 128), xb16.reshape(NP8, 128)],
                      [root1[:16], root1[16:]], row(bias1),
                      row(g1), row(be1), 8, True)
    h1 = h1p.reshape(N, DOUT)

    xj, = _gather16([h1], idx)
    msg = _msg(ea_p, [xj], W2a, row(b2a), W2b, row(b2b), [r2], s2, ex0)
    parts = _scatter(msg, idx, zinit)
    h2p = _post(parts, invp, [h1p], [jnp.pad(root2, ((0, 8), (0, 0)))],
                row(bias2), row(g2), row(be2), 4, False)
    h2 = h2p.reshape(N, DOUT)

    xj, = _gather16([h2], idx)
    msg = _msg(ea_p, [xj], W3a, row(b3a), W3b, row(b3b), [r3], s3, ex0)
    parts = _scatter(msg, idx, zinit)
    h3p = _post(parts, invp, [h2p], [jnp.pad(root3, ((0, 12), (0, 0)))],
                row(bias3), row(g3), row(be3), 16, False)
    return h3p.reshape(N, DOUT)


# speedup vs baseline: 1.3762x; 1.0083x over previous
import functools

import numpy as np
import jax
import jax.numpy as jnp
from jax import lax
from jax.experimental import pallas as pl
from jax.experimental.pallas import tpu as pltpu
from jax.experimental.pallas import tpu_sc as plsc

N = 10000
E = 160000
NC, NS = 2, 16
NW = NC * NS
K = 125
C = E // (NW * K)
PW = C * K
CPB = 8
NB = C // CPB
NPAD = 10240
RP = NPAD // NS
DOUT = 16
EP = E // 8
EPS = 1e-5
_MESH = dict(core_axis_name="c", subcore_axis_name="s")


def _gather16(tables, idx_w):
    nt = len(tables)
    mesh = plsc.VectorSubcoreMesh(**_MESH)

    @functools.partial(
        pl.kernel,
        out_type=[jax.ShapeDtypeStruct((E, DOUT), jnp.float32)] * nt,
        mesh=mesh,
        compiler_params=pltpu.CompilerParams(use_tc_tiling_on_sc=False),
        scratch_types=[pltpu.VMEM((C, K), jnp.int32)]
        + [pltpu.VMEM((CPB * K, DOUT), jnp.float32)] * nt
        + [pltpu.SemaphoreType.DMA],
    )
    def gk(*refs):
        tabs = refs[:nt]
        idx_hbm = refs[nt]
        outs = refs[nt + 1:2 * nt + 1]
        idx_v = refs[2 * nt + 1]
        bufs = refs[2 * nt + 2:3 * nt + 2]
        sem = refs[3 * nt + 2]
        wid = lax.axis_index("s") * NC + lax.axis_index("c")
        base = wid * PW
        pltpu.sync_copy(idx_hbm.at[0, wid], idx_v)

        @pl.loop(0, NB)
        def _(cc):
            cps = []
            for t in range(CPB):
                for tab, buf in zip(tabs, bufs):
                    cps.append(pltpu.async_copy(
                        tab.at[idx_v.at[cc * CPB + t]],
                        buf.at[pl.ds(t * K, K)], sem))
            for cp in cps:
                cp.wait()
            for buf, out in zip(bufs, outs):
                pltpu.sync_copy(
                    buf, out.at[pl.ds(base + cc * (CPB * K), CPB * K)])

    res = gk(*tables, idx_w)
    if not isinstance(res, (list, tuple)):
        res = [res]
    return [r.reshape(EP, 128) for r in res]


def _scatter(msg_p, idx_w, zinit):
    mesh = plsc.VectorSubcoreMesh(**_MESH)

    @functools.partial(
        pl.kernel,
        out_type=jax.ShapeDtypeStruct((NC, NPAD, DOUT), jnp.float32),
        mesh=mesh,
        compiler_params=pltpu.CompilerParams(use_tc_tiling_on_sc=False),
        scratch_types=[
            pltpu.VMEM((C, K), jnp.int32),
            pltpu.VMEM((PW, DOUT), jnp.float32),
            pltpu.VMEM_SHARED((NPAD, DOUT), jnp.float32),
            pltpu.SemaphoreType.DMA,
        ],
    )
    def sk(msg_hbm, idx_hbm, zero_hbm, out_hbm, idx_v, msg_v, acc_sh, sem):
        cid = lax.axis_index("c")
        sid = lax.axis_index("s")
        wid = sid * NC + cid
        row0 = sid * RP
        pltpu.sync_copy(zero_hbm.at[pl.ds(row0, RP)], acc_sh.at[pl.ds(row0, RP)])
        plsc.subcore_barrier()
        pltpu.sync_copy(msg_hbm.at[pl.ds(wid * PW, PW)], msg_v)
        pltpu.sync_copy(idx_hbm.at[1, wid], idx_v)

        @pl.loop(0, NB)
        def _(cc):
            cps = []
            for t in range(CPB):
                j = cc * CPB + t
                cps.append(pltpu.async_copy(
                    msg_v.at[pl.ds(j * K, K)], acc_sh.at[idx_v.at[j]], sem,
                    add=True))
            for cp in cps:
                cp.wait()

        plsc.subcore_barrier()
        pltpu.sync_copy(acc_sh.at[pl.ds(row0, RP)], out_hbm.at[cid, pl.ds(row0, RP)])

    return sk(msg_p.reshape(E, DOUT), idx_w, zinit)


def _eq_consts():
    eqs_np = np.zeros((8 * 128, DOUT), np.float32)
    for q in range(8):
        for c in range(DOUT):
            eqs_np[q * 128 + q * DOUT + c, c] = 1.0
    eqt_np = np.concatenate(
        [eqs_np[q * 128:(q + 1) * 128].T for q in range(8)], axis=1)
    return jnp.asarray(eqs_np), jnp.asarray(eqt_np)


def _msg(ea_p, xps, Wa, ba, Wb, bb, Rs, Sm, extra, block_e=16000):
    G = E // block_e
    PR = block_e // 8
    Ha = Wa.shape[1]
    nx = len(xps)
    eqs, eqt = _eq_consts()
    f32 = jnp.float32
    dj = functools.partial(jnp.dot, preferred_element_type=f32)
    exp = jnp.tile(extra, (1, 8))

    if Ha <= 64:
        HW = 8 * Ha
        waw = jnp.concatenate(
            [dj(eqs[q * 128:(q + 1) * 128], Wa) for q in range(8)], axis=1)
        rqw = [jnp.concatenate(
            [dj(eqs[q * 128:(q + 1) * 128], r) for q in range(8)], axis=1)
            for r in Rs]
        wbd = jnp.kron(jnp.eye(8, dtype=f32), Wb).astype(jnp.bfloat16)
        sqw = jnp.concatenate(
            [dj(Sm, eqt[:, q * 128:(q + 1) * 128]) for q in range(8)], axis=0)
        baw = jnp.tile(ba, (1, 8))
        dsq = jnp.dot(jnp.diag(bb[0]), Sm, preferred_element_type=f32)
        dsqw = jnp.concatenate(
            [dj(dsq, eqt[:, q * 128:(q + 1) * 128]) for q in range(8)], axis=0)

        def body(*refs):
            ea_ref = refs[0]
            xp_refs = refs[1:1 + nx]
            (waw_ref, baw_ref, wbd_ref) = refs[1 + nx:4 + nx]
            rq_refs = refs[4 + nx:4 + 2 * nx]
            (sqw_ref, dsq_ref, ex_ref, out_ref) = refs[4 + 2 * nx:]
            eap = ea_ref[...]
            h = jnp.maximum(dj(eap, waw_ref[...]) + baw_ref[...], 0.0)
            we = jnp.dot(h.astype(jnp.bfloat16), wbd_ref[...],
                         preferred_element_type=jnp.float32)
            xt = dj(xp_refs[0][...], rq_refs[0][...])
            for t in range(1, nx):
                xt = xt + dj(xp_refs[t][...], rq_refs[t][...])
            out_ref[...] = (ex_ref[...] + dj(we * xt, sqw_ref[...])
                            + dj(xt, dsq_ref[...]))

        full = lambda shape: pl.BlockSpec(shape, lambda i: (0, 0))
        return pl.pallas_call(
            body,
            grid=(G,),
            in_specs=[pl.BlockSpec((PR, 128), lambda i: (i, 0))] * (1 + nx)
            + [full((128, HW)), full((1, HW)), full((HW, HW))]
            + [full((128, HW))] * nx
            + [full((HW, 128)), full((HW, 128)), full((1, 128))],
            out_specs=pl.BlockSpec((PR, 128), lambda i: (i, 0)),
            out_shape=jax.ShapeDtypeStruct((EP, 128), jnp.float32),
        )(ea_p, *xps, waw, baw, wbd, *rqw, sqw, dsqw, exp)

    waq = dj(eqs, Wa)
    rq = [dj(eqs, r) for r in Rs]
    sq = dj(Sm, eqt)

    def body(*refs):
        ea_ref = refs[0]
        xp_refs = refs[1:1 + nx]
        (waq_ref, ba_ref, wb_ref, bb_ref) = refs[1 + nx:5 + nx]
        rq_refs = refs[5 + nx:5 + 2 * nx]
        (sq_ref, ex_ref, out_ref) = refs[5 + 2 * nx:]
        dot = functools.partial(jnp.dot, preferred_element_type=f32)
        eap = ea_ref[...]
        xpv = [r[...] for r in xp_refs]
        acc = ex_ref[...] + jnp.zeros((PR, 128), f32)
        for q in range(8):
            h = jnp.maximum(
                dot(eap, waq_ref[pl.ds(q * 128, 128), :]) + ba_ref[...], 0.0)
            we = jnp.dot(h.astype(jnp.bfloat16), wb_ref[...],
                         preferred_element_type=jnp.float32) + bb_ref[...]
            xt = dot(xpv[0], rq_refs[0][pl.ds(q * 128, 128), :])
            for t in range(1, nx):
                xt = xt + dot(xpv[t], rq_refs[t][pl.ds(q * 128, 128), :])
            acc = acc + dot(we * xt, sq_ref[:, pl.ds(q * 128, 128)])
        out_ref[...] = acc

    full = lambda shape: pl.BlockSpec(shape, lambda i: (0, 0))
    return pl.pallas_call(
        body,
        grid=(G,),
        in_specs=[pl.BlockSpec((PR, 128), lambda i: (i, 0))] * (1 + nx)
        + [full((8 * 128, Ha)), full((1, Ha)), full((Ha, Ha)), full((1, Ha))]
        + [full((8 * 128, Ha))] * nx
        + [full((Ha, 8 * 128)), full((1, 128))],
        out_specs=pl.BlockSpec((PR, 128), lambda i: (i, 0)),
        out_shape=jax.ShapeDtypeStruct((EP, 128), jnp.float32),
    )(ea_p, *xps, waq, ba, Wb.astype(jnp.bfloat16), bb, *rq, sq, exp)


NP8 = N // 8
PPAD = NPAD // 8


def _post(parts, invp_in, xps_cur, roots, bias, g, be, c_out, with_cnt):
    f32 = jnp.float32
    dj = functools.partial(jnp.dot, preferred_element_type=f32)
    eqs, eqt = _eq_consts()
    nt = len(xps_cur)
    csel = jnp.asarray(np.eye(DOUT, c_out, dtype=np.float32))
    ec = dj(eqs, csel)
    erq = [dj(eqs, r) for r in roots]
    poq = jnp.concatenate(
        [eqt[:c_out, q * 128:(q + 1) * 128] for q in range(8)], axis=0)
    pp = parts.reshape(NC * PPAD, 128)
    ins = [pp] + ([] if with_cnt else [invp_in]) + list(xps_cur) \
        + [ec] + erq + [poq, bias, g, be]
    outs = [jax.ShapeDtypeStruct((NP8, 128), jnp.float32)]
    if with_cnt:
        e8 = dj(eqs, jnp.asarray(
            np.eye(DOUT, 1, k=-c_out, dtype=np.float32)))
        ones8 = np.zeros((8, 128), np.float32)
        for q in range(8):
            ones8[q, q * DOUT:(q + 1) * DOUT] = 1.0
        ins += [e8, jnp.asarray(ones8)]
        outs.append(jax.ShapeDtypeStruct((NP8, 128), jnp.float32))

    def body(*refs):
        i = 0
        pp_ref = refs[i]; i += 1
        if not with_cnt:
            invp_ref = refs[i]; i += 1
        xp_refs = refs[i:i + nt]; i += nt
        ec_ref = refs[i]; i += 1
        erq_refs = refs[i:i + nt]; i += nt
        poq_ref, bias_ref, g_ref, be_ref = refs[i:i + 4]; i += 4
        if with_cnt:
            e8_ref, ones_ref = refs[i:i + 2]; i += 2
            out_ref, invp_out = refs[i:i + 2]
        else:
            out_ref = refs[i]
        acc = pp_ref[0:NP8, :] + pp_ref[PPAD:PPAD + NP8, :]
        if not with_cnt:
            acc = acc * invp_ref[...]
        xpv = [r[...] for r in xp_refs]
        hs, invs = [], []
        su = jnp.zeros((1, c_out), f32)
        ssq = jnp.zeros((1, c_out), f32)
        for q in range(8):
            hq = dj(acc, ec_ref[pl.ds(q * 128, 128), :])
            if with_cnt:
                cnt = dj(acc, e8_ref[pl.ds(q * 128, 128), :])
                inv = 1.0 / jnp.maximum(cnt, 1.0)
                invs.append(inv)
                hq = hq * inv
            for t in range(nt):
                hq = hq + dj(xpv[t], erq_refs[t][pl.ds(q * 128, 128), :])
            hq = hq + bias_ref[...]
            hs.append(hq)
            su = su + jnp.sum(hq, axis=0, keepdims=True)
            ssq = ssq + jnp.sum(hq * hq, axis=0, keepdims=True)
        mu = su * (1.0 / N)
        var = ssq * (1.0 / N) - mu * mu
        scale = g_ref[...] * lax.rsqrt(var + EPS)
        shift = be_ref[...] - mu * scale
        out = jnp.zeros((NP8, 128), f32)
        for q in range(8):
            y = jnp.maximum(hs[q] * scale + shift, 0.0)
            out = out + dj(y, poq_ref[pl.ds(q * c_out, c_out), :])
        out_ref[...] = out
        if with_cnt:
            ip = jnp.zeros((NP8, 128), f32)
            for q in range(8):
                ip = ip + dj(invs[q], ones_ref[pl.ds(q, 1), :])
            invp_out[...] = ip

    res = pl.pallas_call(body, out_shape=outs)(*ins)
    return res if with_cnt else res[0]


def _mk_RS(c_in, c_out):
    ha = c_in * c_out
    fp = 32 if c_in == 32 else DOUT
    rm = np.zeros((fp, ha), np.float32)
    sm = np.zeros((ha, DOUT), np.float32)
    for i in range(c_in):
        for o in range(c_out):
            rm[i, i * c_out + o] = 1.0
            sm[i * c_out + o, o] = 1.0
    return jnp.asarray(rm), jnp.asarray(sm)


def kernel(x, edge_index, edge_attr, W1a, b1a, W1b, b1b, root1, bias1, g1, be1,
           W2a, b2a, W2b, b2b, root2, bias2, g2, be2,
           W3a, b3a, W3b, b3b, root3, bias3, g3, be3):
    idx = edge_index.astype(jnp.int32).reshape(2, NW, C, K)
    zinit = jnp.zeros((NPAD, DOUT), jnp.float32)
    ea_p = edge_attr.reshape(EP, 128)

    r1, s1 = _mk_RS(32, 8)
    r2, s2 = _mk_RS(8, 4)
    r3, s3 = _mk_RS(4, 16)
    ex1 = np.zeros((1, DOUT), np.float32)
    ex1[0, 8] = 1.0
    ex1 = jnp.asarray(ex1)
    ex0 = jnp.zeros((1, DOUT), jnp.float32)

    def row(v):
        return v.reshape(1, -1)

    xa16, xb16 = x[:, :16], x[:, 16:]
    xa, xb = _gather16([xa16, xb16], idx)
    msg = _msg(ea_p, [xa, xb], W1a, row(b1a), W1b, row(b1b),
               [r1[:16], r1[16:]], s1, ex1, block_e=32000)
    parts = _scatter(msg, idx, zinit)
    h1p, invp = _post(parts, None,
                      [xa16.reshape(NP8, 128), xb16.reshape(NP8, 128)],
                      [root1[:16], root1[16:]], row(bias1),
                      row(g1), row(be1), 8, True)
    h1 = h1p.reshape(N, DOUT)

    xj, = _gather16([h1], idx)
    msg = _msg(ea_p, [xj], W2a, row(b2a), W2b, row(b2b), [r2], s2, ex0)
    parts = _scatter(msg, idx, zinit)
    h2p = _post(parts, invp, [h1p], [jnp.pad(root2, ((0, 8), (0, 0)))],
                row(bias2), row(g2), row(be2), 4, False)
    h2 = h2p.reshape(N, DOUT)

    xj, = _gather16([h2], idx)
    msg = _msg(ea_p, [xj], W3a, row(b3a), W3b, row(b3b), [r3], s3, ex0)
    parts = _scatter(msg, idx, zinit)
    h3p = _post(parts, invp, [h2p], [jnp.pad(root3, ((0, 12), (0, 0)))],
                row(bias3), row(g3), row(be3), 16, False)
    return h3p.reshape(N, DOUT)
